# baseline (speedup 1.0000x reference)
import functools

import jax
import jax.numpy as jnp
from jax import lax
from jax.experimental import pallas as pl
from jax.experimental.pallas import tpu as pltpu
from jax.experimental.pallas import tpu_sc as plsc

VOCAB = 1000000
EMB = 64
NC = 2
NS = 16
NW = NC * NS
NI = 4096
NJ = 200
IW = NI // NW
IG = 2
NGROUPS = IW // IG


def _gather_kernel(idx_hbm, tab_hbm, out_hbm, idx_v, buf_a, buf_b, sem_a, sem_b):
    c = lax.axis_index("c")
    s = lax.axis_index("s")
    w = s * NC + c
    i0 = w * IW
    pltpu.sync_copy(idx_hbm.at[pl.ds(i0, IW), pl.ds(0, NJ)], idx_v)

    def fire(g, buf, sem):
        for k in range(IG):
            pltpu.async_copy(tab_hbm.at[idx_v.at[g * IG + k]], buf.at[k], sem)

    def drain(buf, sem):
        for k in range(IG):
            pltpu.make_async_copy(tab_hbm.at[idx_v.at[0]], buf.at[k], sem).wait()

    def writeback(g, buf):
        pltpu.sync_copy(buf, out_hbm.at[pl.ds(i0 + g * IG, IG), pl.ds(0, NJ), pl.ds(0, EMB)])

    fire(0, buf_a, sem_a)

    def body(p, carry):
        g = 2 * p
        drain(buf_a, sem_a)
        fire(g + 1, buf_b, sem_b)
        writeback(g, buf_a)
        drain(buf_b, sem_b)
        fire(g + 2, buf_a, sem_a)
        writeback(g + 1, buf_b)
        return carry

    lax.fori_loop(0, NGROUPS // 2 - 1, body, 0)

    g = NGROUPS - 2
    drain(buf_a, sem_a)
    fire(g + 1, buf_b, sem_b)
    writeback(g, buf_a)
    drain(buf_b, sem_b)
    writeback(g + 1, buf_b)


@jax.jit
def _run(idx, tvectors):
    mesh = plsc.VectorSubcoreMesh(core_axis_name="c", subcore_axis_name="s")
    k = functools.partial(
        pl.kernel,
        mesh=mesh,
        out_type=jax.ShapeDtypeStruct((NI, NJ, EMB), jnp.float32),
        scratch_types=[
            pltpu.VMEM((IW, NJ), jnp.int32),
            pltpu.VMEM((IG, NJ, EMB), jnp.float32),
            pltpu.VMEM((IG, NJ, EMB), jnp.float32),
            pltpu.SemaphoreType.DMA,
            pltpu.SemaphoreType.DMA,
        ],
        compiler_params=pltpu.CompilerParams(use_tc_tiling_on_sc=False),
    )(_gather_kernel)
    return k(idx, tvectors)


def kernel(data, tvectors):
    return _run(data.astype(jnp.int32), tvectors)
